# baseline (speedup 1.0000x reference)
import functools

import jax
import jax.numpy as jnp
from jax import lax
from jax.experimental import pallas as pl
from jax.experimental.pallas import tpu as pltpu
from jax.experimental.pallas import tpu_sc as plsc

B = 16384
D = 64
NC = 2
NS = 16
NW = NC * NS
BPW = B // NW
CHUNK = 128
NCHUNK = BPW // CHUNK


def _sc_cf_dot(x_hbm, user_hbm, item_hbm, out_hbm,
               x_v, idx_u, idx_i, rows_u, rows_i, out_v, sem):
    cid = lax.axis_index("c")
    sid = lax.axis_index("s")
    wid = sid * NC + cid
    base = wid * BPW

    pltpu.sync_copy(x_hbm.at[pl.ds(base * 2, BPW * 2)], x_v)

    iota16 = lax.iota(jnp.int32, 16)
    for g in range(BPW // 16):
        even16 = (iota16 + (g * 16)) * 2
        c = (g * 16) // CHUNK
        off = (g * 16) % CHUNK
        idx_u[c, pl.ds(off, 16)] = plsc.load_gather(x_v, [even16])
        idx_i[c, pl.ds(off, 16)] = plsc.load_gather(x_v, [even16 + 1])

    copies = []
    for c in range(NCHUNK):
        copies.append(pltpu.async_copy(
            user_hbm.at[idx_u.at[c]], rows_u.at[pl.ds(c * CHUNK, CHUNK), :], sem))
        copies.append(pltpu.async_copy(
            item_hbm.at[idx_i.at[c]], rows_i.at[pl.ds(c * CHUNK, CHUNK), :], sem))
    for cp in copies:
        cp.wait()

    lane15 = iota16 == 15
    def body(r, carry):
        acc = rows_u[r, pl.ds(0, 16)] * rows_i[r, pl.ds(0, 16)]
        for k in range(1, D // 16):
            acc = acc + rows_u[r, pl.ds(k * 16, 16)] * rows_i[r, pl.ds(k * 16, 16)]
        cs = plsc.cumsum(acc)
        plsc.store_scatter(out_v, [jnp.full((16,), r, jnp.int32)], cs, mask=lane15)
        return carry
    lax.fori_loop(0, BPW, body, 0)

    pltpu.sync_copy(out_v, out_hbm.at[pl.ds(base, BPW)])


@jax.jit
def kernel(x, user_emb, item_emb):
    mesh = plsc.VectorSubcoreMesh(core_axis_name="c", subcore_axis_name="s")
    f = functools.partial(
        pl.kernel,
        mesh=mesh,
        out_type=jax.ShapeDtypeStruct((B,), jnp.float32),
        scratch_types=[
            pltpu.VMEM((BPW * 2,), jnp.int32),
            pltpu.VMEM((NCHUNK, CHUNK), jnp.int32),
            pltpu.VMEM((NCHUNK, CHUNK), jnp.int32),
            pltpu.VMEM((BPW, D), jnp.float32),
            pltpu.VMEM((BPW, D), jnp.float32),
            pltpu.VMEM((BPW,), jnp.float32),
            pltpu.SemaphoreType.DMA,
        ],
        compiler_params=pltpu.CompilerParams(
            needs_layout_passes=False, use_tc_tiling_on_sc=False),
    )(_sc_cf_dot)
    return f(x.astype(jnp.int32).reshape(-1), user_emb, item_emb)
